# Initial kernel scaffold; baseline (speedup 1.0000x reference)
import jax
import jax.numpy as jnp
from jax.experimental import pallas as pl


def kernel(x, edge_index, prop_nums, W0, b0, Wc, bc):
    raise NotImplementedError("write your pallas kernel here")


# speedup vs baseline: 4.8197x; 4.8197x over previous
import functools

import jax
import jax.numpy as jnp
from jax import lax
from jax.experimental import pallas as pl
from jax.experimental.pallas import tpu as pltpu
from jax.experimental.pallas import tpu_sc as plsc

N_NODES = 10000
NP = 10240
D = 128
C_OUT = 10
CP = 16
E_RAW = 320000
E_FULL = E_RAW + N_NODES
CH = 5504
E_PAD = 330240
NCHUNK = E_PAD // CH
NTILES = 32
FPT = D // NTILES
ESH = E_PAD // NTILES
COLB = 1024

_mesh = plsc.VectorSubcoreMesh(core_axis_name="c", subcore_axis_name="s")
_sc_params = pltpu.CompilerParams(needs_layout_passes=False)


def _wid():
    return lax.axis_index("c") * 16 + lax.axis_index("s")


@functools.partial(
    pl.kernel,
    out_type=jax.ShapeDtypeStruct((NTILES, NP), jnp.float32),
    mesh=_mesh,
    compiler_params=_sc_params,
    scratch_types=[
        pltpu.VMEM((1, NP), jnp.float32),
        pltpu.VMEM((ESH,), jnp.int32),
        pltpu.SemaphoreType.DMA,
    ],
)
def _deg_kernel(dst_hbm, hist_hbm, hist_t, ebuf, sem):
    wid = _wid()
    pltpu.async_copy(dst_hbm.at[pl.ds(wid * ESH, ESH)], ebuf, sem).wait()

    @pl.loop(0, NP, step=16)
    def _(i):
        hist_t[0, pl.ds(i, 16)] = jnp.zeros((16,), jnp.float32)

    z16 = jnp.zeros((16,), jnp.int32)
    one16 = jnp.ones((16,), jnp.float32)

    @pl.loop(0, ESH, step=16)
    def _(e):
        d16 = ebuf[pl.ds(e, 16)]
        plsc.addupdate_scatter(hist_t, [z16, d16], one16)

    pltpu.sync_copy(hist_t, hist_hbm.at[pl.ds(wid, 1), :])


@functools.partial(
    pl.kernel,
    out_type=jax.ShapeDtypeStruct((D, NP), jnp.float32),
    mesh=_mesh,
    compiler_params=_sc_params,
    scratch_types=[
        pltpu.VMEM((FPT, NP), jnp.float32),
        pltpu.VMEM((FPT, NP), jnp.float32),
        pltpu.VMEM((1, NP), jnp.float32),
        pltpu.VMEM((2, CH), jnp.int32),
        pltpu.VMEM((2, CH), jnp.int32),
        pltpu.VMEM((16,), jnp.int32),
        pltpu.SemaphoreType.DMA,
        pltpu.SemaphoreType.DMA,
        pltpu.SemaphoreType.DMA,
    ],
)
def _prop_kernel(u0_hbm, src_hbm, dst_hbm, dinv_hbm, ns_hbm, out_hbm,
                 u_t, acc_t, dinv_t, sbuf, dbuf, nsv, sem_a, sem_b, sem_m):
    wid = _wid()
    f0 = wid * FPT
    pltpu.async_copy(u0_hbm.at[pl.ds(f0, FPT), :], u_t, sem_m).wait()
    pltpu.async_copy(dinv_hbm, dinv_t, sem_m).wait()
    pltpu.async_copy(ns_hbm, nsv, sem_m).wait()
    nsteps = jnp.max(nsv[...])

    fvecs = [jnp.full((16,), f, jnp.int32) for f in range(FPT)]

    def _issue(ci, buf, sem):
        pltpu.async_copy(src_hbm.at[pl.ds(ci * CH, CH)], sbuf.at[buf], sem)
        pltpu.async_copy(dst_hbm.at[pl.ds(ci * CH, CH)], dbuf.at[buf], sem)

    def _wait(ci, buf, sem):
        pltpu.make_async_copy(
            src_hbm.at[pl.ds(ci * CH, CH)], sbuf.at[buf], sem).wait()
        pltpu.make_async_copy(
            dst_hbm.at[pl.ds(ci * CH, CH)], dbuf.at[buf], sem).wait()

    def _process(buf):
        @pl.loop(0, CH, step=16)
        def _(e):
            s16 = sbuf[buf, pl.ds(e, 16)]
            d16 = dbuf[buf, pl.ds(e, 16)]
            for f in range(FPT):
                v = plsc.load_gather(u_t, [fvecs[f], s16])
                plsc.addupdate_scatter(acc_t, [fvecs[f], d16], v)

    def _step(_, carry):
        @pl.loop(0, NP, step=16)
        def _(i):
            for f in range(FPT):
                acc_t[f, pl.ds(i, 16)] = jnp.zeros((16,), jnp.float32)

        _issue(0, 0, sem_a)

        @pl.loop(0, NCHUNK, step=2)
        def _(ci):
            _issue(ci + 1, 1, sem_b)
            _wait(ci, 0, sem_a)
            _process(0)

            @pl.when(ci + 2 < NCHUNK)
            def _():
                _issue(ci + 2, 0, sem_a)

            _wait(ci + 1, 1, sem_b)
            _process(1)

        @pl.loop(0, NP, step=16)
        def _(i):
            dv = dinv_t[0, pl.ds(i, 16)]
            for f in range(FPT):
                u_t[f, pl.ds(i, 16)] = acc_t[f, pl.ds(i, 16)] * dv

        return carry

    lax.fori_loop(0, nsteps, _step, 0)
    pltpu.sync_copy(u_t, out_hbm.at[pl.ds(f0, FPT), :])


@functools.partial(
    pl.kernel,
    out_type=jax.ShapeDtypeStruct((CP, NP), jnp.float32),
    mesh=_mesh,
    compiler_params=_sc_params,
    scratch_types=[
        pltpu.VMEM((1, NP), jnp.float32),
        pltpu.VMEM((1, NP), jnp.float32),
        pltpu.VMEM((1, NP), jnp.float32),
        pltpu.VMEM((2, CH), jnp.int32),
        pltpu.VMEM((2, CH), jnp.int32),
        pltpu.SemaphoreType.DMA,
        pltpu.SemaphoreType.DMA,
        pltpu.SemaphoreType.DMA,
    ],
)
def _final_kernel(z_hbm, src_hbm, dst_hbm, disq_hbm, out_hbm,
                  z_t, acc_t, disq_t, sbuf, dbuf, sem_a, sem_b, sem_m):
    wid = _wid()

    @pl.when(wid < CP)
    def _():
        pltpu.async_copy(z_hbm.at[pl.ds(wid, 1), :], z_t, sem_m).wait()
        pltpu.async_copy(disq_hbm, disq_t, sem_m).wait()

        @pl.loop(0, NP, step=16)
        def _(i):
            acc_t[0, pl.ds(i, 16)] = jnp.zeros((16,), jnp.float32)

        z16 = jnp.zeros((16,), jnp.int32)

        def _issue(ci, buf, sem):
            pltpu.async_copy(src_hbm.at[pl.ds(ci * CH, CH)], sbuf.at[buf], sem)
            pltpu.async_copy(dst_hbm.at[pl.ds(ci * CH, CH)], dbuf.at[buf], sem)

        def _wait(ci, buf, sem):
            pltpu.make_async_copy(
                src_hbm.at[pl.ds(ci * CH, CH)], sbuf.at[buf], sem).wait()
            pltpu.make_async_copy(
                dst_hbm.at[pl.ds(ci * CH, CH)], dbuf.at[buf], sem).wait()

        def _process(buf):
            @pl.loop(0, CH, step=16)
            def _(e):
                s16 = sbuf[buf, pl.ds(e, 16)]
                d16 = dbuf[buf, pl.ds(e, 16)]
                v = plsc.load_gather(z_t, [z16, s16])
                plsc.addupdate_scatter(acc_t, [z16, d16], v)

        _issue(0, 0, sem_a)

        @pl.loop(0, NCHUNK, step=2)
        def _(ci):
            _issue(ci + 1, 1, sem_b)
            _wait(ci, 0, sem_a)
            _process(0)

            @pl.when(ci + 2 < NCHUNK)
            def _():
                _issue(ci + 2, 0, sem_a)

            _wait(ci + 1, 1, sem_b)
            _process(1)

        @pl.loop(0, NP, step=16)
        def _(i):
            acc_t[0, pl.ds(i, 16)] = acc_t[0, pl.ds(i, 16)] * disq_t[0, pl.ds(i, 16)]

        pltpu.sync_copy(acc_t, out_hbm.at[pl.ds(wid, 1), :])


def _mm0_body(w_ref, x_ref, b_ref, o_ref):
    o_ref[...] = lax.dot_general(
        w_ref[...], x_ref[...], (((0,), (1,)), ((), ())),
        preferred_element_type=jnp.float32) + b_ref[...]


_mm0 = pl.pallas_call(
    _mm0_body,
    grid=(NP // COLB,),
    in_specs=[
        pl.BlockSpec((D, D), lambda i: (0, 0)),
        pl.BlockSpec((COLB, D), lambda i: (i, 0)),
        pl.BlockSpec((D, 1), lambda i: (0, 0)),
    ],
    out_specs=pl.BlockSpec((D, COLB), lambda i: (0, i)),
    out_shape=jax.ShapeDtypeStruct((D, NP), jnp.float32),
)


def _scale_body(h_ref, hist_ref, u0_ref, dinv_ref, dsq_ref, disq_ref):
    deg = jnp.sum(hist_ref[...], axis=0, keepdims=True)
    pos = deg > 0
    dinv_ref[...] = jnp.where(pos, 1.0 / deg, 0.0)
    sq = jnp.sqrt(deg)
    dsq_ref[...] = sq
    disq = jnp.where(pos, 1.0 / sq, 0.0)
    disq_ref[...] = disq
    u0_ref[...] = h_ref[...] * disq


_scale = pl.pallas_call(
    _scale_body,
    grid=(NP // COLB,),
    in_specs=[
        pl.BlockSpec((D, COLB), lambda i: (0, i)),
        pl.BlockSpec((NTILES, COLB), lambda i: (0, i)),
    ],
    out_specs=[
        pl.BlockSpec((D, COLB), lambda i: (0, i)),
        pl.BlockSpec((1, COLB), lambda i: (0, i)),
        pl.BlockSpec((1, COLB), lambda i: (0, i)),
        pl.BlockSpec((1, COLB), lambda i: (0, i)),
    ],
    out_shape=[
        jax.ShapeDtypeStruct((D, NP), jnp.float32),
        jax.ShapeDtypeStruct((1, NP), jnp.float32),
        jax.ShapeDtypeStruct((1, NP), jnp.float32),
        jax.ShapeDtypeStruct((1, NP), jnp.float32),
    ],
)


def _clf_body(wt_ref, u_ref, dsq_ref, disq_ref, bc_ref, z_ref):
    y = jnp.maximum(u_ref[...], 0.0) * dsq_ref[...]
    z = lax.dot_general(
        wt_ref[...], y, (((1,), (0,)), ((), ())),
        preferred_element_type=jnp.float32)
    z_ref[...] = (z + bc_ref[...]) * disq_ref[...]


_clf = pl.pallas_call(
    _clf_body,
    grid=(NP // COLB,),
    in_specs=[
        pl.BlockSpec((CP, D), lambda i: (0, 0)),
        pl.BlockSpec((D, COLB), lambda i: (0, i)),
        pl.BlockSpec((1, COLB), lambda i: (0, i)),
        pl.BlockSpec((1, COLB), lambda i: (0, i)),
        pl.BlockSpec((CP, 1), lambda i: (0, 0)),
    ],
    out_specs=pl.BlockSpec((CP, COLB), lambda i: (0, i)),
    out_shape=jax.ShapeDtypeStruct((CP, NP), jnp.float32),
)


def kernel(x, edge_index, prop_nums, W0, b0, Wc, bc):
    src = edge_index[0].astype(jnp.int32)
    dst = edge_index[1].astype(jnp.int32)
    loop_idx = jnp.arange(N_NODES, dtype=jnp.int32)
    padv = N_NODES + (jnp.arange(E_PAD - E_FULL, dtype=jnp.int32) % 16)
    src_f = jnp.concatenate([src, loop_idx, padv])
    dst_f = jnp.concatenate([dst, loop_idx, padv])

    x_pad = jnp.pad(x, ((0, NP - N_NODES), (0, 0)))
    b0c = b0.reshape(D, 1)
    wct = jnp.pad(Wc, ((0, 0), (0, CP - C_OUT))).T
    bcp = jnp.pad(bc, (0, CP - C_OUT)).reshape(CP, 1)
    ns_arr = jnp.full((16,), prop_nums, jnp.int32)

    hist = _deg_kernel(dst_f)
    h0t = _mm0(W0, x_pad, b0c)
    u0, dinv, dsq, disq = _scale(h0t, hist)
    u30 = _prop_kernel(u0, src_f, dst_f, dinv, ns_arr)
    z2 = _clf(wct, u30, dsq, disq, bcp)
    outt = _final_kernel(z2, src_f, dst_f, disq)
    return outt[:C_OUT, :N_NODES].T


# speedup vs baseline: 12.8257x; 2.6611x over previous
import functools

import jax
import jax.numpy as jnp
from jax import lax
from jax.experimental import pallas as pl
from jax.experimental.pallas import tpu as pltpu
from jax.experimental.pallas import tpu_sc as plsc

N_NODES = 10000
NP = 10240
D = 128
C_OUT = 10
CP = 16
E_RAW = 320000
E_FULL = E_RAW + N_NODES
CH = 5504
E_PAD = 330240
NCHUNK = E_PAD // CH
NTILES = 32
FPT = D // NTILES
ESH = E_PAD // NTILES
COLB = 1024

_mesh = plsc.VectorSubcoreMesh(core_axis_name="c", subcore_axis_name="s")
_sc_params = pltpu.CompilerParams(needs_layout_passes=False)


def _wid():
    return lax.axis_index("c") * 16 + lax.axis_index("s")


def _unpack(pk):
    s16 = lax.bitwise_and(pk, jnp.int32(0xFFFF))
    d16 = lax.shift_right_logical(pk, jnp.int32(16))
    return s16, d16


@functools.partial(
    pl.kernel,
    out_type=jax.ShapeDtypeStruct((NTILES, NP), jnp.float32),
    mesh=_mesh,
    compiler_params=_sc_params,
    scratch_types=[
        pltpu.VMEM((NP,), jnp.float32),
        pltpu.VMEM((ESH,), jnp.int32),
        pltpu.SemaphoreType.DMA,
    ],
)
def _deg_kernel(edge_hbm, hist_hbm, hist_t, ebuf, sem):
    wid = _wid()
    pltpu.async_copy(edge_hbm.at[pl.ds(wid * ESH, ESH)], ebuf, sem).wait()

    @pl.loop(0, NP, step=16, unroll=4)
    def _(i):
        hist_t[pl.ds(i, 16)] = jnp.zeros((16,), jnp.float32)

    one16 = jnp.ones((16,), jnp.float32)

    @pl.loop(0, ESH, step=16, unroll=2)
    def _(e):
        pk = ebuf[pl.ds(e, 16)]
        _, d16 = _unpack(pk)
        plsc.addupdate_scatter(hist_t, [d16], one16)

    pltpu.sync_copy(hist_t, hist_hbm.at[wid])


@functools.partial(
    pl.kernel,
    out_type=jax.ShapeDtypeStruct((D, NP), jnp.float32),
    mesh=_mesh,
    compiler_params=_sc_params,
    scratch_types=[
        [pltpu.VMEM((NP,), jnp.float32)] * FPT,
        [pltpu.VMEM((NP,), jnp.float32)] * FPT,
        pltpu.VMEM((NP,), jnp.float32),
        pltpu.VMEM((2, CH), jnp.int32),
        pltpu.VMEM((16,), jnp.int32),
        pltpu.SemaphoreType.DMA,
        pltpu.SemaphoreType.DMA,
        pltpu.SemaphoreType.DMA,
    ],
)
def _prop_kernel(u0_hbm, edge_hbm, dinv_hbm, ns_hbm, out_hbm,
                 u_refs, acc_refs, dinv_t, ebuf, nsv, sem_a, sem_b, sem_m):
    wid = _wid()
    f0 = wid * FPT
    for f in range(FPT):
        pltpu.async_copy(u0_hbm.at[f0 + f], u_refs[f], sem_m)
    pltpu.async_copy(dinv_hbm, dinv_t, sem_m)
    pltpu.async_copy(ns_hbm, nsv, sem_m)
    for f in range(FPT):
        pltpu.make_async_copy(u0_hbm.at[f0 + f], u_refs[f], sem_m).wait()
    pltpu.make_async_copy(dinv_hbm, dinv_t, sem_m).wait()
    pltpu.make_async_copy(ns_hbm, nsv, sem_m).wait()
    nsteps = jnp.max(nsv[...])

    def _issue(ci, buf, sem):
        pltpu.async_copy(edge_hbm.at[pl.ds(ci * CH, CH)], ebuf.at[buf], sem)

    def _wait(ci, buf, sem):
        pltpu.make_async_copy(
            edge_hbm.at[pl.ds(ci * CH, CH)], ebuf.at[buf], sem).wait()

    def _process(buf):
        @plsc.parallel_loop(0, CH, 16, unroll=4)
        def _(e):
            pk = ebuf[buf, pl.ds(e, 16)]
            s16, d16 = _unpack(pk)
            vs = [plsc.load_gather(u_refs[f], [s16]) for f in range(FPT)]
            for f in range(FPT):
                plsc.addupdate_scatter(acc_refs[f], [d16], vs[f])

    def _step(_, carry):
        @pl.loop(0, NP, step=16, unroll=4)
        def _(i):
            for f in range(FPT):
                acc_refs[f][pl.ds(i, 16)] = jnp.zeros((16,), jnp.float32)

        _issue(0, 0, sem_a)

        @pl.loop(0, NCHUNK, step=2)
        def _(ci):
            _issue(ci + 1, 1, sem_b)
            _wait(ci, 0, sem_a)
            _process(0)

            @pl.when(ci + 2 < NCHUNK)
            def _():
                _issue(ci + 2, 0, sem_a)

            _wait(ci + 1, 1, sem_b)
            _process(1)

        @pl.loop(0, NP, step=16, unroll=4)
        def _(i):
            dv = dinv_t[pl.ds(i, 16)]
            for f in range(FPT):
                u_refs[f][pl.ds(i, 16)] = acc_refs[f][pl.ds(i, 16)] * dv

        return carry

    lax.fori_loop(0, nsteps, _step, 0)
    for f in range(FPT):
        pltpu.async_copy(u_refs[f], out_hbm.at[f0 + f], sem_m)
    for f in range(FPT):
        pltpu.make_async_copy(u_refs[f], out_hbm.at[f0 + f], sem_m).wait()


@functools.partial(
    pl.kernel,
    out_type=jax.ShapeDtypeStruct((CP, NP), jnp.float32),
    mesh=_mesh,
    compiler_params=_sc_params,
    scratch_types=[
        pltpu.VMEM((NP,), jnp.float32),
        pltpu.VMEM((NP,), jnp.float32),
        pltpu.VMEM((NP,), jnp.float32),
        pltpu.VMEM((2, CH), jnp.int32),
        pltpu.SemaphoreType.DMA,
        pltpu.SemaphoreType.DMA,
        pltpu.SemaphoreType.DMA,
    ],
)
def _final_kernel(z_hbm, edge_hbm, disq_hbm, out_hbm,
                  z_t, acc_t, disq_t, ebuf, sem_a, sem_b, sem_m):
    wid = _wid()

    @pl.when(wid < CP)
    def _():
        pltpu.async_copy(z_hbm.at[wid], z_t, sem_m).wait()
        pltpu.async_copy(disq_hbm, disq_t, sem_m).wait()

        @pl.loop(0, NP, step=16, unroll=4)
        def _(i):
            acc_t[pl.ds(i, 16)] = jnp.zeros((16,), jnp.float32)

        def _issue(ci, buf, sem):
            pltpu.async_copy(edge_hbm.at[pl.ds(ci * CH, CH)], ebuf.at[buf], sem)

        def _wait(ci, buf, sem):
            pltpu.make_async_copy(
                edge_hbm.at[pl.ds(ci * CH, CH)], ebuf.at[buf], sem).wait()

        def _process(buf):
            @pl.loop(0, CH, step=16, unroll=2)
            def _(e):
                pk = ebuf[buf, pl.ds(e, 16)]
                s16, d16 = _unpack(pk)
                v = plsc.load_gather(z_t, [s16])
                plsc.addupdate_scatter(acc_t, [d16], v)

        _issue(0, 0, sem_a)

        @pl.loop(0, NCHUNK, step=2)
        def _(ci):
            _issue(ci + 1, 1, sem_b)
            _wait(ci, 0, sem_a)
            _process(0)

            @pl.when(ci + 2 < NCHUNK)
            def _():
                _issue(ci + 2, 0, sem_a)

            _wait(ci + 1, 1, sem_b)
            _process(1)

        @pl.loop(0, NP, step=16, unroll=4)
        def _(i):
            acc_t[pl.ds(i, 16)] = acc_t[pl.ds(i, 16)] * disq_t[pl.ds(i, 16)]

        pltpu.sync_copy(acc_t, out_hbm.at[wid])


def _mm0_body(w_ref, x_ref, b_ref, o_ref):
    o_ref[...] = lax.dot_general(
        w_ref[...], x_ref[...], (((0,), (1,)), ((), ())),
        preferred_element_type=jnp.float32) + b_ref[...]


_mm0 = pl.pallas_call(
    _mm0_body,
    grid=(NP // COLB,),
    in_specs=[
        pl.BlockSpec((D, D), lambda i: (0, 0)),
        pl.BlockSpec((COLB, D), lambda i: (i, 0)),
        pl.BlockSpec((D, 1), lambda i: (0, 0)),
    ],
    out_specs=pl.BlockSpec((D, COLB), lambda i: (0, i)),
    out_shape=jax.ShapeDtypeStruct((D, NP), jnp.float32),
)


def _scale_body(h_ref, hist_ref, u0_ref, dinv_ref, dsq_ref, disq_ref):
    deg = jnp.sum(hist_ref[...], axis=0, keepdims=True)
    pos = deg > 0
    dinv_ref[...] = jnp.where(pos, 1.0 / deg, 0.0)
    sq = jnp.sqrt(deg)
    dsq_ref[...] = sq
    disq = jnp.where(pos, 1.0 / sq, 0.0)
    disq_ref[...] = disq
    u0_ref[...] = h_ref[...] * disq


_scale = pl.pallas_call(
    _scale_body,
    grid=(NP // COLB,),
    in_specs=[
        pl.BlockSpec((D, COLB), lambda i: (0, i)),
        pl.BlockSpec((NTILES, COLB), lambda i: (0, i)),
    ],
    out_specs=[
        pl.BlockSpec((D, COLB), lambda i: (0, i)),
        pl.BlockSpec((1, COLB), lambda i: (0, i)),
        pl.BlockSpec((1, COLB), lambda i: (0, i)),
        pl.BlockSpec((1, COLB), lambda i: (0, i)),
    ],
    out_shape=[
        jax.ShapeDtypeStruct((D, NP), jnp.float32),
        jax.ShapeDtypeStruct((1, NP), jnp.float32),
        jax.ShapeDtypeStruct((1, NP), jnp.float32),
        jax.ShapeDtypeStruct((1, NP), jnp.float32),
    ],
)


def _clf_body(wt_ref, u_ref, dsq_ref, disq_ref, bc_ref, z_ref):
    y = jnp.maximum(u_ref[...], 0.0) * dsq_ref[...]
    z = lax.dot_general(
        wt_ref[...], y, (((1,), (0,)), ((), ())),
        preferred_element_type=jnp.float32)
    z_ref[...] = (z + bc_ref[...]) * disq_ref[...]


_clf = pl.pallas_call(
    _clf_body,
    grid=(NP // COLB,),
    in_specs=[
        pl.BlockSpec((CP, D), lambda i: (0, 0)),
        pl.BlockSpec((D, COLB), lambda i: (0, i)),
        pl.BlockSpec((1, COLB), lambda i: (0, i)),
        pl.BlockSpec((1, COLB), lambda i: (0, i)),
        pl.BlockSpec((CP, 1), lambda i: (0, 0)),
    ],
    out_specs=pl.BlockSpec((CP, COLB), lambda i: (0, i)),
    out_shape=jax.ShapeDtypeStruct((CP, NP), jnp.float32),
)


def kernel(x, edge_index, prop_nums, W0, b0, Wc, bc):
    src = edge_index[0].astype(jnp.int32)
    dst = edge_index[1].astype(jnp.int32)
    loop_idx = jnp.arange(N_NODES, dtype=jnp.int32)
    padv = N_NODES + (jnp.arange(E_PAD - E_FULL, dtype=jnp.int32) % 16)
    src_f = jnp.concatenate([src, loop_idx, padv])
    dst_f = jnp.concatenate([dst, loop_idx, padv])
    epk = jnp.bitwise_or(src_f, jnp.left_shift(dst_f, 16))

    x_pad = jnp.pad(x, ((0, NP - N_NODES), (0, 0)))
    b0c = b0.reshape(D, 1)
    wct = jnp.pad(Wc, ((0, 0), (0, CP - C_OUT))).T
    bcp = jnp.pad(bc, (0, CP - C_OUT)).reshape(CP, 1)
    ns_arr = jnp.full((16,), prop_nums, jnp.int32)

    hist = _deg_kernel(epk)
    h0t = _mm0(W0, x_pad, b0c)
    u0, dinv, dsq, disq = _scale(h0t, hist)
    u30 = _prop_kernel(u0, epk, dinv.reshape(NP), ns_arr)
    z2 = _clf(wct, u30, dsq, disq, bcp)
    outt = _final_kernel(z2, epk, disq.reshape(NP))
    return outt[:C_OUT, :N_NODES].T


# speedup vs baseline: 13.2165x; 1.0305x over previous
import functools

import jax
import jax.numpy as jnp
from jax import lax
from jax.experimental import pallas as pl
from jax.experimental.pallas import tpu as pltpu
from jax.experimental.pallas import tpu_sc as plsc

N_NODES = 10000
NP = 10240
D = 128
C_OUT = 10
CP = 16
E_RAW = 320000
E_FULL = E_RAW + N_NODES
CH = 16512
E_PAD = 330240
NCHUNK = E_PAD // CH
NTILES = 32
FPT = D // NTILES
ESH = E_PAD // NTILES
COLB = 1024

_mesh = plsc.VectorSubcoreMesh(core_axis_name="c", subcore_axis_name="s")
_sc_params = pltpu.CompilerParams(needs_layout_passes=False)


def _wid():
    return lax.axis_index("c") * 16 + lax.axis_index("s")


def _unpack(pk):
    s16 = lax.bitwise_and(pk, jnp.int32(0xFFFF))
    d16 = lax.shift_right_logical(pk, jnp.int32(16))
    return s16, d16


@functools.partial(
    pl.kernel,
    out_type=jax.ShapeDtypeStruct((NTILES, NP), jnp.float32),
    mesh=_mesh,
    compiler_params=_sc_params,
    scratch_types=[
        pltpu.VMEM((NP,), jnp.float32),
        pltpu.VMEM((ESH,), jnp.int32),
        pltpu.SemaphoreType.DMA,
    ],
)
def _deg_kernel(edge_hbm, hist_hbm, hist_t, ebuf, sem):
    wid = _wid()
    pltpu.async_copy(edge_hbm.at[pl.ds(wid * ESH, ESH)], ebuf, sem).wait()

    @pl.loop(0, NP, step=16, unroll=4)
    def _(i):
        hist_t[pl.ds(i, 16)] = jnp.zeros((16,), jnp.float32)

    one16 = jnp.ones((16,), jnp.float32)

    @plsc.parallel_loop(0, ESH, 16, unroll=4)
    def _(e):
        pk = ebuf[pl.ds(e, 16)]
        _, d16 = _unpack(pk)
        plsc.addupdate_scatter(hist_t, [d16], one16)

    pltpu.sync_copy(hist_t, hist_hbm.at[wid])


@functools.partial(
    pl.kernel,
    out_type=jax.ShapeDtypeStruct((D, NP), jnp.float32),
    mesh=_mesh,
    compiler_params=_sc_params,
    scratch_types=[
        [pltpu.VMEM((NP,), jnp.float32)] * FPT,
        [pltpu.VMEM((NP,), jnp.float32)] * FPT,
        pltpu.VMEM((NP,), jnp.float32),
        pltpu.VMEM((2, CH), jnp.int32),
        pltpu.VMEM((16,), jnp.int32),
        pltpu.SemaphoreType.DMA,
        pltpu.SemaphoreType.DMA,
        pltpu.SemaphoreType.DMA,
    ],
)
def _prop_kernel(u0_hbm, edge_hbm, dinv_hbm, ns_hbm, out_hbm,
                 u_refs, acc_refs, dinv_t, ebuf, nsv, sem_a, sem_b, sem_m):
    wid = _wid()
    f0 = wid * FPT
    for f in range(FPT):
        pltpu.async_copy(u0_hbm.at[f0 + f], u_refs[f], sem_m)
    pltpu.async_copy(dinv_hbm, dinv_t, sem_m)
    pltpu.async_copy(ns_hbm, nsv, sem_m)
    for f in range(FPT):
        pltpu.make_async_copy(u0_hbm.at[f0 + f], u_refs[f], sem_m).wait()
    pltpu.make_async_copy(dinv_hbm, dinv_t, sem_m).wait()
    pltpu.make_async_copy(ns_hbm, nsv, sem_m).wait()
    nsteps = jnp.max(nsv[...])

    def _issue(ci, buf, sem):
        pltpu.async_copy(edge_hbm.at[pl.ds(ci * CH, CH)], ebuf.at[buf], sem)

    def _wait(ci, buf, sem):
        pltpu.make_async_copy(
            edge_hbm.at[pl.ds(ci * CH, CH)], ebuf.at[buf], sem).wait()

    def _process(buf):
        @plsc.parallel_loop(0, CH, 16, unroll=4)
        def _(e):
            pk = ebuf[buf, pl.ds(e, 16)]
            s16, d16 = _unpack(pk)
            vs = [plsc.load_gather(u_refs[f], [s16]) for f in range(FPT)]
            for f in range(FPT):
                plsc.addupdate_scatter(acc_refs[f], [d16], vs[f])

    def _step(_, carry):
        @pl.loop(0, NP, step=16, unroll=4)
        def _(i):
            for f in range(FPT):
                acc_refs[f][pl.ds(i, 16)] = jnp.zeros((16,), jnp.float32)

        _issue(0, 0, sem_a)

        @pl.loop(0, NCHUNK, step=2)
        def _(ci):
            _issue(ci + 1, 1, sem_b)
            _wait(ci, 0, sem_a)
            _process(0)

            @pl.when(ci + 2 < NCHUNK)
            def _():
                _issue(ci + 2, 0, sem_a)

            _wait(ci + 1, 1, sem_b)
            _process(1)

        @pl.loop(0, NP, step=16, unroll=4)
        def _(i):
            dv = dinv_t[pl.ds(i, 16)]
            for f in range(FPT):
                u_refs[f][pl.ds(i, 16)] = acc_refs[f][pl.ds(i, 16)] * dv

        return carry

    lax.fori_loop(0, nsteps, _step, 0)
    for f in range(FPT):
        pltpu.async_copy(u_refs[f], out_hbm.at[f0 + f], sem_m)
    for f in range(FPT):
        pltpu.make_async_copy(u_refs[f], out_hbm.at[f0 + f], sem_m).wait()


@functools.partial(
    pl.kernel,
    out_type=jax.ShapeDtypeStruct((CP, NP), jnp.float32),
    mesh=_mesh,
    compiler_params=_sc_params,
    scratch_types=[
        pltpu.VMEM((NP,), jnp.float32),
        pltpu.VMEM((NP,), jnp.float32),
        pltpu.VMEM((NP,), jnp.float32),
        pltpu.VMEM((2, CH), jnp.int32),
        pltpu.SemaphoreType.DMA,
        pltpu.SemaphoreType.DMA,
        pltpu.SemaphoreType.DMA,
    ],
)
def _final_kernel(z_hbm, edge_hbm, disq_hbm, out_hbm,
                  z_t, acc_t, disq_t, ebuf, sem_a, sem_b, sem_m):
    wid = _wid()

    @pl.when(wid < CP)
    def _():
        pltpu.async_copy(z_hbm.at[wid], z_t, sem_m).wait()
        pltpu.async_copy(disq_hbm, disq_t, sem_m).wait()

        @pl.loop(0, NP, step=16, unroll=4)
        def _(i):
            acc_t[pl.ds(i, 16)] = jnp.zeros((16,), jnp.float32)

        def _issue(ci, buf, sem):
            pltpu.async_copy(edge_hbm.at[pl.ds(ci * CH, CH)], ebuf.at[buf], sem)

        def _wait(ci, buf, sem):
            pltpu.make_async_copy(
                edge_hbm.at[pl.ds(ci * CH, CH)], ebuf.at[buf], sem).wait()

        def _process(buf):
            @plsc.parallel_loop(0, CH, 16, unroll=4)
            def _(e):
                pk = ebuf[buf, pl.ds(e, 16)]
                s16, d16 = _unpack(pk)
                v = plsc.load_gather(z_t, [s16])
                plsc.addupdate_scatter(acc_t, [d16], v)

        _issue(0, 0, sem_a)

        @pl.loop(0, NCHUNK, step=2)
        def _(ci):
            _issue(ci + 1, 1, sem_b)
            _wait(ci, 0, sem_a)
            _process(0)

            @pl.when(ci + 2 < NCHUNK)
            def _():
                _issue(ci + 2, 0, sem_a)

            _wait(ci + 1, 1, sem_b)
            _process(1)

        @pl.loop(0, NP, step=16, unroll=4)
        def _(i):
            acc_t[pl.ds(i, 16)] = acc_t[pl.ds(i, 16)] * disq_t[pl.ds(i, 16)]

        pltpu.sync_copy(acc_t, out_hbm.at[wid])


def _mm0_body(w_ref, x_ref, b_ref, o_ref):
    o_ref[...] = lax.dot_general(
        w_ref[...], x_ref[...], (((0,), (1,)), ((), ())),
        preferred_element_type=jnp.float32) + b_ref[...]


_mm0 = pl.pallas_call(
    _mm0_body,
    grid=(NP // COLB,),
    in_specs=[
        pl.BlockSpec((D, D), lambda i: (0, 0)),
        pl.BlockSpec((COLB, D), lambda i: (i, 0)),
        pl.BlockSpec((D, 1), lambda i: (0, 0)),
    ],
    out_specs=pl.BlockSpec((D, COLB), lambda i: (0, i)),
    out_shape=jax.ShapeDtypeStruct((D, NP), jnp.float32),
)


def _scale_body(h_ref, hist_ref, u0_ref, dinv_ref, dsq_ref, disq_ref):
    deg = jnp.sum(hist_ref[...], axis=0, keepdims=True)
    pos = deg > 0
    dinv_ref[...] = jnp.where(pos, 1.0 / deg, 0.0)
    sq = jnp.sqrt(deg)
    dsq_ref[...] = sq
    disq = jnp.where(pos, 1.0 / sq, 0.0)
    disq_ref[...] = disq
    u0_ref[...] = h_ref[...] * disq


_scale = pl.pallas_call(
    _scale_body,
    grid=(NP // COLB,),
    in_specs=[
        pl.BlockSpec((D, COLB), lambda i: (0, i)),
        pl.BlockSpec((NTILES, COLB), lambda i: (0, i)),
    ],
    out_specs=[
        pl.BlockSpec((D, COLB), lambda i: (0, i)),
        pl.BlockSpec((1, COLB), lambda i: (0, i)),
        pl.BlockSpec((1, COLB), lambda i: (0, i)),
        pl.BlockSpec((1, COLB), lambda i: (0, i)),
    ],
    out_shape=[
        jax.ShapeDtypeStruct((D, NP), jnp.float32),
        jax.ShapeDtypeStruct((1, NP), jnp.float32),
        jax.ShapeDtypeStruct((1, NP), jnp.float32),
        jax.ShapeDtypeStruct((1, NP), jnp.float32),
    ],
)


def _clf_body(wt_ref, u_ref, dsq_ref, disq_ref, bc_ref, z_ref):
    y = jnp.maximum(u_ref[...], 0.0) * dsq_ref[...]
    z = lax.dot_general(
        wt_ref[...], y, (((1,), (0,)), ((), ())),
        preferred_element_type=jnp.float32)
    z_ref[...] = (z + bc_ref[...]) * disq_ref[...]


_clf = pl.pallas_call(
    _clf_body,
    grid=(NP // COLB,),
    in_specs=[
        pl.BlockSpec((CP, D), lambda i: (0, 0)),
        pl.BlockSpec((D, COLB), lambda i: (0, i)),
        pl.BlockSpec((1, COLB), lambda i: (0, i)),
        pl.BlockSpec((1, COLB), lambda i: (0, i)),
        pl.BlockSpec((CP, 1), lambda i: (0, 0)),
    ],
    out_specs=pl.BlockSpec((CP, COLB), lambda i: (0, i)),
    out_shape=jax.ShapeDtypeStruct((CP, NP), jnp.float32),
)


def kernel(x, edge_index, prop_nums, W0, b0, Wc, bc):
    src = edge_index[0].astype(jnp.int32)
    dst = edge_index[1].astype(jnp.int32)
    loop_idx = jnp.arange(N_NODES, dtype=jnp.int32)
    padv = N_NODES + (jnp.arange(E_PAD - E_FULL, dtype=jnp.int32) % 16)
    src_f = jnp.concatenate([src, loop_idx, padv])
    dst_f = jnp.concatenate([dst, loop_idx, padv])
    epk = jnp.bitwise_or(src_f, jnp.left_shift(dst_f, 16))

    x_pad = jnp.pad(x, ((0, NP - N_NODES), (0, 0)))
    b0c = b0.reshape(D, 1)
    wct = jnp.pad(Wc, ((0, 0), (0, CP - C_OUT))).T
    bcp = jnp.pad(bc, (0, CP - C_OUT)).reshape(CP, 1)
    ns_arr = jnp.full((16,), prop_nums, jnp.int32)

    hist = _deg_kernel(epk)
    h0t = _mm0(W0, x_pad, b0c)
    u0, dinv, dsq, disq = _scale(h0t, hist)
    u30 = _prop_kernel(u0, epk, dinv.reshape(NP), ns_arr)
    z2 = _clf(wct, u30, dsq, disq, bcp)
    outt = _final_kernel(z2, epk, disq.reshape(NP))
    return outt[:C_OUT, :N_NODES].T
